# baseline (speedup 1.0000x reference)
import functools

import jax
import jax.numpy as jnp
from jax import lax
from jax.experimental import pallas as pl
from jax.experimental.pallas import tpu as pltpu
from jax.experimental.pallas import tpu_sc as plsc

B = 16384
K = 64
NUM_WORKERS = 32
BPW = B // NUM_WORKERS
LANES = 16


def _sc_kernel_body(f0_hbm, f1_hbm, f2_hbm, i0_hbm, i1_hbm, i2_hbm, out_hbm,
                    i0_v, i1_v, i2_v, r0_v, r1_v, r2_v, out_v, sem):
    wid = lax.axis_index("s") * 2 + lax.axis_index("c")
    base = wid * BPW

    pltpu.sync_copy(i0_hbm.at[pl.ds(base, BPW)], i0_v)
    pltpu.sync_copy(i1_hbm.at[pl.ds(base, BPW)], i1_v)
    pltpu.sync_copy(i2_hbm.at[pl.ds(base, BPW)], i2_v)

    c0 = pltpu.async_copy(f0_hbm.at[i0_v], r0_v, sem)
    c1 = pltpu.async_copy(f1_hbm.at[i1_v], r1_v, sem)
    c2 = pltpu.async_copy(f2_hbm.at[i2_v], r2_v, sem)
    c0.wait()
    c1.wait()
    c2.wait()

    lane_iota = lax.iota(jnp.int32, LANES)

    def body(g, carry):
        vec = jnp.zeros((LANES,), jnp.float32)
        gbase = g * jnp.int32(LANES)
        for l in range(LANES):
            b = gbase + jnp.int32(l)
            acc = None
            for j in range(K // LANES):
                sl = pl.ds(j * LANES, LANES)
                p = r0_v[b, sl] * r1_v[b, sl] * r2_v[b, sl]
                acc = p if acc is None else acc + p
            vec = jnp.where(lane_iota == jnp.int32(l), jnp.sum(acc), vec)
        out_v[pl.ds(gbase, LANES)] = vec
        return carry

    lax.fori_loop(jnp.int32(0), jnp.int32(BPW // LANES), body, jnp.int32(0))

    pltpu.sync_copy(out_v, out_hbm.at[pl.ds(base, BPW)])


@jax.jit
def _run(f0, f1, f2, i0, i1, i2):
    mesh = plsc.VectorSubcoreMesh(core_axis_name="c", subcore_axis_name="s")
    kern = functools.partial(
        pl.kernel,
        out_type=jax.ShapeDtypeStruct((B,), jnp.float32),
        mesh=mesh,
        scratch_types=[
            pltpu.VMEM((BPW,), jnp.int32),
            pltpu.VMEM((BPW,), jnp.int32),
            pltpu.VMEM((BPW,), jnp.int32),
            pltpu.VMEM((BPW, K), jnp.float32),
            pltpu.VMEM((BPW, K), jnp.float32),
            pltpu.VMEM((BPW, K), jnp.float32),
            pltpu.VMEM((BPW,), jnp.float32),
            pltpu.SemaphoreType.DMA,
        ],
        compiler_params=pltpu.CompilerParams(
            needs_layout_passes=False, use_tc_tiling_on_sc=False),
    )(_sc_kernel_body)
    return kern(f0, f1, f2, i0, i1, i2)


def _hi_plane(f):
    from jax.experimental import layout as jex_layout
    hi = f.astype(jnp.float32)
    return jex_layout.with_layout_constraint(hi, jex_layout.Layout((1, 0)))


def kernel(indices, F0, F1, F2):
    idx = indices.astype(jnp.int32)
    out = _run(_hi_plane(F0), _hi_plane(F1), _hi_plane(F2),
               idx[:, 0], idx[:, 1], idx[:, 2])
    return out.astype(jnp.float64)


# speedup vs baseline: 1.0311x; 1.0311x over previous
import functools

import jax
import jax.numpy as jnp
from jax import lax
from jax.experimental import pallas as pl
from jax.experimental.pallas import tpu as pltpu
from jax.experimental.pallas import tpu_sc as plsc

B = 16384
K = 64
NUM_WORKERS = 32
BPW = B // NUM_WORKERS
LANES = 16


def _sc_kernel_body(f0_hbm, f1_hbm, f2_hbm, i0_hbm, i1_hbm, i2_hbm, out_hbm,
                    i0_v, i1_v, i2_v, r0_v, r1_v, r2_v, out_v, sem):
    wid = lax.axis_index("s") * 2 + lax.axis_index("c")
    base = wid * BPW

    pltpu.sync_copy(i0_hbm.at[pl.ds(base, BPW)], i0_v)
    pltpu.sync_copy(i1_hbm.at[pl.ds(base, BPW)], i1_v)
    pltpu.sync_copy(i2_hbm.at[pl.ds(base, BPW)], i2_v)

    c0 = pltpu.async_copy(f0_hbm.at[i0_v], r0_v, sem)
    c1 = pltpu.async_copy(f1_hbm.at[i1_v], r1_v, sem)
    c2 = pltpu.async_copy(f2_hbm.at[i2_v], r2_v, sem)
    c0.wait()
    c1.wait()
    c2.wait()

    lane_iota = lax.iota(jnp.int32, LANES)

    def body(g, carry):
        vec = jnp.zeros((LANES,), jnp.float32)
        gbase = g * jnp.int32(LANES)
        for l in range(LANES):
            b = gbase + jnp.int32(l)
            acc = None
            for j in range(K // LANES):
                sl = pl.ds(j * LANES, LANES)
                p = (plsc.bitcast(r0_v[b, sl], jnp.float32)
                     * plsc.bitcast(r1_v[b, sl], jnp.float32)
                     * plsc.bitcast(r2_v[b, sl], jnp.float32))
                acc = p if acc is None else acc + p
            vec = jnp.where(lane_iota == jnp.int32(l), jnp.sum(acc), vec)
        out_v[pl.ds(gbase, LANES)] = vec
        return carry

    lax.fori_loop(jnp.int32(0), jnp.int32(BPW // LANES), body, jnp.int32(0))

    pltpu.sync_copy(out_v, out_hbm.at[pl.ds(base, BPW)])


@jax.jit
def _run(f0, f1, f2, i0, i1, i2):
    mesh = plsc.VectorSubcoreMesh(core_axis_name="c", subcore_axis_name="s")
    kern = functools.partial(
        pl.kernel,
        out_type=jax.ShapeDtypeStruct((B,), jnp.float32),
        mesh=mesh,
        scratch_types=[
            pltpu.VMEM((BPW,), jnp.int32),
            pltpu.VMEM((BPW,), jnp.int32),
            pltpu.VMEM((BPW,), jnp.int32),
            pltpu.VMEM((BPW, K), jnp.int32),
            pltpu.VMEM((BPW, K), jnp.int32),
            pltpu.VMEM((BPW, K), jnp.int32),
            pltpu.VMEM((BPW,), jnp.float32),
            pltpu.SemaphoreType.DMA,
        ],
        compiler_params=pltpu.CompilerParams(
            needs_layout_passes=False, use_tc_tiling_on_sc=False),
    )(_sc_kernel_body)
    return kern(f0, f1, f2, i0, i1, i2)


def _hi_plane(f):
    from jax.experimental import layout as jex_layout
    hi = f.astype(jnp.float32)
    hi = jex_layout.with_layout_constraint(hi, jex_layout.Layout((1, 0)))
    return lax.bitcast_convert_type(hi, jnp.int32)


def kernel(indices, F0, F1, F2):
    idx = indices.astype(jnp.int32)
    out = _run(_hi_plane(F0), _hi_plane(F1), _hi_plane(F2),
               idx[:, 0], idx[:, 1], idx[:, 2])
    return out.astype(jnp.float64)
